# Initial kernel scaffold; baseline (speedup 1.0000x reference)
import jax
import jax.numpy as jnp
from jax.experimental import pallas as pl


def kernel(x, offsets):
    raise NotImplementedError("write your pallas kernel here")


# speedup vs baseline: 7.4493x; 7.4493x over previous
import functools

import jax
import jax.numpy as jnp
from jax import lax
from jax.experimental import pallas as pl
from jax.experimental.pallas import tpu as pltpu
from jax.experimental.pallas import tpu_sc as plsc

L = 16
NC = 2
NS = 16
R = 2


def _jitter_body(B, D, T, x_hbm, off_hbm, out_hbm,
                 offv, idxv, x00, x01, x10, x11, o00, o01, o10, o11,
                 isem0, isem1, osem0, osem1):
    c = lax.axis_index("c")
    s = lax.axis_index("s")
    w = s * NC + c
    b = w // (NC * NS // B)
    half = w % (NC * NS // B)
    rows = D // (NC * NS // B)
    d0 = half * rows

    xb = ((x00, x01), (x10, x11))
    ob = ((o00, o01), (o10, o11))
    isems = (isem0, isem1)
    osems = (osem0, osem1)
    nstep = rows // R

    def in_cps(g, slot):
        return [pltpu.make_async_copy(
            x_hbm.at[b, d0 + g * R + r, :], xb[slot][r], isems[slot])
            for r in range(R)]

    def out_cps(g, slot):
        return [pltpu.make_async_copy(
            ob[slot][r], out_hbm.at[b, d0 + g * R + r, :], osems[slot])
            for r in range(R)]

    pltpu.sync_copy(off_hbm.at[b], offv)
    for cp in in_cps(0, 0):
        cp.start()

    def mk_idx(i, carry):
        base = i * L
        off = offv[pl.ds(base, L)]
        idx = lax.iota(jnp.int32, L) + (base - 1) + off
        idx = jnp.minimum(jnp.maximum(idx, 0), T - 1)
        idxv[pl.ds(base, L)] = idx
        return carry

    lax.fori_loop(0, T // L, mk_idx, 0)

    def outer(i, carry):
        for k in range(2):
            g = i * 2 + k
            slot = k
            nxt = (k + 1) % 2

            @pl.when(g + 1 < nstep)
            def _():
                for cp in in_cps(g + 1, nxt):
                    cp.start()

            for cp in in_cps(g, slot):
                cp.wait()

            @pl.when(g >= 2)
            def _():
                for cp in out_cps(g - 2, slot):
                    cp.wait()

            def chunk(j, cc):
                base = j * L
                tv = idxv[pl.ds(base, L)]
                for r in range(R):
                    ob[slot][r][pl.ds(base, L)] = plsc.load_gather(
                        xb[slot][r], [tv])
                return cc

            lax.fori_loop(0, T // L, chunk, 0)
            for cp in out_cps(g, slot):
                cp.start()
        return carry

    lax.fori_loop(0, nstep // 2, outer, 0)
    for cp in out_cps(nstep - 2, 0):
        cp.wait()
    for cp in out_cps(nstep - 1, 1):
        cp.wait()


def kernel(x, offsets):
    B, D, T = x.shape
    mesh = plsc.VectorSubcoreMesh(core_axis_name="c", subcore_axis_name="s")
    f = pl.kernel(
        functools.partial(_jitter_body, B, D, T),
        out_type=jax.ShapeDtypeStruct(x.shape, x.dtype),
        mesh=mesh,
        compiler_params=pltpu.CompilerParams(needs_layout_passes=False),
        scratch_types=(
            [pltpu.VMEM((T,), jnp.int32)] * 2 +
            [pltpu.VMEM((T,), jnp.float32)] * 8 +
            [pltpu.SemaphoreType.DMA] * 4
        ),
    )
    return f(x, offsets)


# speedup vs baseline: 20.7679x; 2.7879x over previous
import functools

import jax
import jax.numpy as jnp
from jax import lax
from jax.experimental import pallas as pl
from jax.experimental.pallas import tpu as pltpu
from jax.experimental.pallas import tpu_sc as plsc

L = 16
NC = 2
NS = 16
R = 2


def _jitter_body(B, D, T, x_hbm, off_hbm, out_hbm,
                 offv, idxv, x00, x01, x10, x11, o00, o01, o10, o11,
                 isem0, isem1, osem0, osem1):
    c = lax.axis_index("c")
    s = lax.axis_index("s")
    w = s * NC + c
    b = w // (NC * NS // B)
    half = w % (NC * NS // B)
    rows = D // (NC * NS // B)
    d0 = half * rows

    xb = ((x00, x01), (x10, x11))
    ob = ((o00, o01), (o10, o11))
    isems = (isem0, isem1)
    osems = (osem0, osem1)
    nstep = rows // R

    def in_cps(g, slot):
        return [pltpu.make_async_copy(
            x_hbm.at[b, d0 + g * R + r, :], xb[slot][r], isems[slot])
            for r in range(R)]

    def out_cps(g, slot):
        return [pltpu.make_async_copy(
            ob[slot][r], out_hbm.at[b, d0 + g * R + r, :], osems[slot])
            for r in range(R)]

    pltpu.sync_copy(off_hbm.at[b], offv)
    for cp in in_cps(0, 0):
        cp.start()

    @plsc.parallel_loop(0, T // L, unroll=4)
    def mk_idx(i):
        base = i * L
        off = offv[pl.ds(base, L)]
        idx = lax.iota(jnp.int32, L) + (base - 1) + off
        idx = jnp.minimum(jnp.maximum(idx, 0), T - 1)
        idxv[pl.ds(base, L)] = idx

    def outer(i, carry):
        for k in range(2):
            g = i * 2 + k
            slot = k
            nxt = (k + 1) % 2

            @pl.when(g + 1 < nstep)
            def _():
                for cp in in_cps(g + 1, nxt):
                    cp.start()

            for cp in in_cps(g, slot):
                cp.wait()

            @pl.when(g >= 2)
            def _():
                for cp in out_cps(g - 2, slot):
                    cp.wait()

            @plsc.parallel_loop(0, T // L, unroll=8)
            def chunk(j):
                base = j * L
                tv = idxv[pl.ds(base, L)]
                for r in range(R):
                    ob[slot][r][pl.ds(base, L)] = plsc.load_gather(
                        xb[slot][r], [tv])
            for cp in out_cps(g, slot):
                cp.start()
        return carry

    lax.fori_loop(0, nstep // 2, outer, 0)
    for cp in out_cps(nstep - 2, 0):
        cp.wait()
    for cp in out_cps(nstep - 1, 1):
        cp.wait()


def kernel(x, offsets):
    B, D, T = x.shape
    mesh = plsc.VectorSubcoreMesh(core_axis_name="c", subcore_axis_name="s")
    f = pl.kernel(
        functools.partial(_jitter_body, B, D, T),
        out_type=jax.ShapeDtypeStruct(x.shape, x.dtype),
        mesh=mesh,
        compiler_params=pltpu.CompilerParams(needs_layout_passes=False),
        scratch_types=(
            [pltpu.VMEM((T,), jnp.int32)] * 2 +
            [pltpu.VMEM((T,), jnp.float32)] * 8 +
            [pltpu.SemaphoreType.DMA] * 4
        ),
    )
    return f(x, offsets)
